# baseline (speedup 1.0000x reference)
import functools

import jax
import jax.numpy as jnp
from jax import lax
from jax.experimental import pallas as pl
from jax.experimental.pallas import tpu as pltpu
from jax.experimental.pallas import tpu_sc as plsc

N = 10000
E = 320000
D = 128
L = 16

NC = 2
NS = 16

EPT = E // NS
CH = 80
NCH = EPT // CH
SS = 10
NSC = NCH // SS

RPT = 640
RPT_LAST = N - 15 * RPT
NB = 10
BN = N // NB


def _sigmoid(v):
    return 1.0 / (1.0 + jnp.exp(-v))


def _dense_body(x_ref, wl_ref, bl_ref, wlg_ref, blg_ref, wi_ref, wgi_ref,
                wo_ref, wgo_ref, xloop_ref, ain_ref, gin_ref, aout_ref,
                gout_ref):
    xb = x_ref[...]
    xl = jnp.dot(xb, wl_ref[...].T, preferred_element_type=jnp.float32) + bl_ref[...]
    gl = _sigmoid(jnp.dot(xl, wlg_ref[...], preferred_element_type=jnp.float32) + blg_ref[0, 0])
    xloop_ref[...] = gl * xl
    ai = jnp.dot(jnp.dot(xb, wi_ref[...], preferred_element_type=jnp.float32),
                 wi_ref[...], preferred_element_type=jnp.float32)
    ain_ref[...] = ai
    gin_ref[...] = jnp.dot(ai, wgi_ref[...], preferred_element_type=jnp.float32)
    ao = jnp.dot(jnp.dot(xb, wo_ref[...], preferred_element_type=jnp.float32),
                 wo_ref[...], preferred_element_type=jnp.float32)
    aout_ref[...] = ao
    gout_ref[...] = jnp.dot(ao, wgo_ref[...], preferred_element_type=jnp.float32)


def _dense(x, W_loop, b_loop, W_loop_g, b_loop_g, W_dir_in, W_dir_g_in,
           W_dir_out, W_dir_g_out):
    full = lambda shape: pl.BlockSpec(shape, lambda i: (0, 0))
    blk = lambda: pl.BlockSpec((BN, D), lambda i: (i, 0))
    nd = jax.ShapeDtypeStruct((N, D), jnp.float32)
    return pl.pallas_call(
        _dense_body,
        grid=(NB,),
        in_specs=[blk(), full((D, D)), full((1, D)), full((D, D)),
                  full((1, 1)), full((D, D)), full((D, D)), full((D, D)),
                  full((D, D))],
        out_specs=[blk(), blk(), blk(), blk(), blk()],
        out_shape=[nd, nd, nd, nd, nd],
    )(x, W_loop, b_loop.reshape(1, D), jnp.broadcast_to(W_loop_g.T, (D, D)),
      b_loop_g.reshape(1, 1), W_dir_in, jnp.broadcast_to(W_dir_g_in, (D, D)),
      W_dir_out, jnp.broadcast_to(W_dir_g_out, (D, D)))


def _edge_body(a2_hbm, g_hbm, c_hbm, b_hbm, sl_hbm, dst_hbm, znd_hbm,
               out_hbm, acc, gsp, ctab, btab, ib, dstb, gb2, sb2, gargb,
               gs0, gs1, as0, as1, ss0, ss1):
    c = lax.axis_index("c")
    s = lax.axis_index("s")
    rbase = s * RPT
    @pl.when(s < 15)
    def _():
        pltpu.sync_copy(znd_hbm.at[pl.ds(rbase, RPT)],
                        acc.at[pl.ds(rbase, RPT)])

    @pl.when(s == 15)
    def _():
        pltpu.sync_copy(znd_hbm.at[pl.ds(15 * RPT, RPT_LAST)],
                        acc.at[pl.ds(15 * RPT, RPT_LAST)])
    pltpu.sync_copy(c_hbm.at[pl.ds(c * L, L)], ctab)
    pltpu.sync_copy(b_hbm.at[c], btab)

    @pl.when(s == 0)
    def _():
        pltpu.sync_copy(g_hbm, gsp)

    plsc.subcore_barrier()

    iota16 = lax.iota(jnp.int32, 16)
    gsems = (gs0, gs1)
    asems = (as0, as1)
    ssems = (ss0, ss1)

    def gissue(i, b):
        idx = ib.at[i, pl.ds(0, CH)]
        pltpu.async_copy(a2_hbm.at[idx], gb2.at[b], gsems[b])
        pltpu.async_copy(gsp.at[idx], gargb.at[b], asems[b])

    def gwait(b):
        idx = ib.at[0, pl.ds(0, CH)]
        pltpu.make_async_copy(a2_hbm.at[idx], gb2.at[b], gsems[b]).wait()
        pltpu.make_async_copy(gsp.at[idx], gargb.at[b], asems[b]).wait()

    def sissue(i, b):
        pltpu.async_copy(sb2.at[b], acc.at[dstb.at[i]], ssems[b], add=True)

    def swait(b):
        pltpu.make_async_copy(sb2.at[b], acc.at[dstb.at[0]], ssems[b]).wait()

    def do_par(fn, b):
        @pl.when(b == 0)
        def _():
            fn(0)

        @pl.when(b == 1)
        def _():
            fn(1)

    def super_body(u, carry):
        @pl.when(u > 0)
        def _():
            swait(0)
            swait(1)

        tidx = (c * NS + s) * NSC + u
        pltpu.sync_copy(sl_hbm.at[tidx], ib)
        pltpu.sync_copy(dst_hbm.at[tidx], dstb)
        gissue(0, 0)
        gissue(1, 1)

        def step(i, carry2):
            b = i % 2
            do_par(gwait, b)

            @pl.when(i >= 2)
            def _():
                do_par(swait, b)

            def group(g, carry3):
                l16 = ib[i, pl.ds(CH + g * 16, 16)]
                garg = gargb[b, pl.ds(g * 16, 16)]
                gate = _sigmoid(garg + plsc.load_gather(ctab, [l16]))
                e16 = g * 16 + iota16
                b16 = jnp.zeros((16,), jnp.int32) + b
                for f in range(D):
                    f16 = jnp.full((16,), f, jnp.int32)
                    v = plsc.load_gather(gb2, [b16, e16, f16])
                    bia = plsc.load_gather(btab, [l16, f16])
                    plsc.store_scatter(sb2, [b16, e16, f16], (v + bia) * gate)
                return carry3

            lax.fori_loop(0, CH // 16, group, 0)
            do_par(lambda bb: sissue(i, bb), b)

            @pl.when(i <= SS - 3)
            def _():
                do_par(lambda bb: gissue(i + 2, bb), b)

            return carry2

        lax.fori_loop(0, SS, step, 0)
        return carry

    lax.fori_loop(0, NSC, super_body, 0)
    swait(0)
    swait(1)
    plsc.subcore_barrier()

    @pl.when(s < 15)
    def _():
        pltpu.sync_copy(acc.at[pl.ds(rbase, RPT)],
                        out_hbm.at[c, pl.ds(rbase, RPT)])

    @pl.when(s == 15)
    def _():
        pltpu.sync_copy(acc.at[pl.ds(15 * RPT, RPT_LAST)],
                        out_hbm.at[c, pl.ds(15 * RPT, RPT_LAST)])


def _edges(a2, g_all, c_all, b_all, sl_all, dst_all, znd):
    mesh = plsc.VectorSubcoreMesh(core_axis_name="c", subcore_axis_name="s")
    f = functools.partial(
        pl.kernel,
        out_type=jax.ShapeDtypeStruct((NC, N, D), jnp.float32),
        mesh=mesh,
        scratch_types=[
            pltpu.VMEM_SHARED((N, D), jnp.float32),
            pltpu.VMEM_SHARED((2 * N,), jnp.float32),
            pltpu.VMEM((L,), jnp.float32),
            pltpu.VMEM((L, D), jnp.float32),
            pltpu.VMEM((SS, 2 * CH), jnp.int32),
            pltpu.VMEM((SS, CH), jnp.int32),
            pltpu.VMEM((2, CH, D), jnp.float32),
            pltpu.VMEM((2, CH, D), jnp.float32),
            pltpu.VMEM((2, CH), jnp.float32),
            pltpu.SemaphoreType.DMA,
            pltpu.SemaphoreType.DMA,
            pltpu.SemaphoreType.DMA,
            pltpu.SemaphoreType.DMA,
            pltpu.SemaphoreType.DMA,
            pltpu.SemaphoreType.DMA,
        ],
        compiler_params=pltpu.CompilerParams(needs_layout_passes=False),
    )(_edge_body)
    return f(a2, g_all, c_all, b_all, sl_all, dst_all, znd)


def _final_body(xl_ref, p0_ref, p1_ref, o_ref):
    o_ref[...] = jnp.maximum(xl_ref[...] + p0_ref[...] + p1_ref[...], 0.0)


def _final(x_loop, p0, p1):
    blk = pl.BlockSpec((BN, D), lambda i: (i, 0))
    return pl.pallas_call(
        _final_body,
        grid=(NB,),
        in_specs=[blk, blk, blk],
        out_specs=blk,
        out_shape=jax.ShapeDtypeStruct((N, D), jnp.float32),
    )(x_loop, p0, p1)


def kernel(x, edge_index, edge_label, W_loop, b_loop, W_loop_g, b_loop_g,
           W_dir_in, b_lab_in, W_dir_g_in, b_lab_g_in,
           W_dir_out, b_lab_out, W_dir_g_out, b_lab_g_out):
    src = edge_index[0]
    dst = edge_index[1]

    x_loop, a_in, g_in, a_out, g_out = _dense(
        x, W_loop, b_loop, W_loop_g, b_loop_g, W_dir_in, W_dir_g_in,
        W_dir_out, W_dir_g_out)

    c_in = (b_lab_in @ W_dir_g_in).reshape(L) + b_lab_g_in.reshape(L)
    c_out = (b_lab_out @ W_dir_g_out).reshape(L) + b_lab_g_out.reshape(L)

    a2 = jnp.concatenate([a_in, a_out], axis=0)
    g_all = jnp.concatenate([g_in[:, 0], g_out[:, 0]])
    c_all = jnp.concatenate([c_in, c_out])
    b_all = jnp.stack([b_lab_in, b_lab_out])
    gi = jnp.concatenate([src, dst + N])
    si = jnp.concatenate([dst, src])
    gi5 = gi.reshape(NC * NS * NSC, SS, 1, CH)
    lab5 = jnp.broadcast_to(edge_label[None], (NC, E)).reshape(
        NC * NS * NSC, SS, 1, CH)
    sl_all = jnp.concatenate([gi5, lab5], axis=2).reshape(
        NC * NS * NSC, SS, 2 * CH)
    dst_all = si.reshape(NC * NS * NSC, SS, CH)
    znd = jnp.zeros((N, D), jnp.float32)

    parts = _edges(a2, g_all, c_all, b_all, sl_all, dst_all, znd)
    return _final(x_loop, parts[0], parts[1])


# speedup vs baseline: 3.9212x; 3.9212x over previous
import functools

import jax
import jax.numpy as jnp
from jax import lax
from jax.experimental import pallas as pl
from jax.experimental.pallas import tpu as pltpu
from jax.experimental.pallas import tpu_sc as plsc

N = 10000
E = 320000
D = 128
L = 16

NC = 2
NS = 16

EPT = E // NS
CH = 80
NCH = EPT // CH
SS = 10
NSC = NCH // SS

RPT = 640
RPT_LAST = N - 15 * RPT
NB = 10
BN = N // NB


def _sigmoid(v):
    return 1.0 / (1.0 + jnp.exp(-v))


def _dense_body(x_ref, wl_ref, bl_ref, wlg_ref, blg_ref, wi_ref, wgi_ref,
                wo_ref, wgo_ref, xloop_ref, ain_ref, gin_ref, aout_ref,
                gout_ref):
    xb = x_ref[...]
    xl = jnp.dot(xb, wl_ref[...].T, preferred_element_type=jnp.float32) + bl_ref[...]
    gl = _sigmoid(jnp.dot(xl, wlg_ref[...], preferred_element_type=jnp.float32) + blg_ref[0, 0])
    xloop_ref[...] = gl * xl
    ai = jnp.dot(jnp.dot(xb, wi_ref[...], preferred_element_type=jnp.float32),
                 wi_ref[...], preferred_element_type=jnp.float32)
    ain_ref[...] = ai
    gin_ref[...] = jnp.dot(ai, wgi_ref[...], preferred_element_type=jnp.float32)
    ao = jnp.dot(jnp.dot(xb, wo_ref[...], preferred_element_type=jnp.float32),
                 wo_ref[...], preferred_element_type=jnp.float32)
    aout_ref[...] = ao
    gout_ref[...] = jnp.dot(ao, wgo_ref[...], preferred_element_type=jnp.float32)


def _dense(x, W_loop, b_loop, W_loop_g, b_loop_g, W_dir_in, W_dir_g_in,
           W_dir_out, W_dir_g_out):
    full = lambda shape: pl.BlockSpec(shape, lambda i: (0, 0))
    blk = lambda: pl.BlockSpec((BN, D), lambda i: (i, 0))
    nd = jax.ShapeDtypeStruct((N, D), jnp.float32)
    return pl.pallas_call(
        _dense_body,
        grid=(NB,),
        in_specs=[blk(), full((D, D)), full((1, D)), full((D, D)),
                  full((1, 1)), full((D, D)), full((D, D)), full((D, D)),
                  full((D, D))],
        out_specs=[blk(), blk(), blk(), blk(), blk()],
        out_shape=[nd, nd, nd, nd, nd],
    )(x, W_loop, b_loop.reshape(1, D), jnp.broadcast_to(W_loop_g.T, (D, D)),
      b_loop_g.reshape(1, 1), W_dir_in, jnp.broadcast_to(W_dir_g_in, (D, D)),
      W_dir_out, jnp.broadcast_to(W_dir_g_out, (D, D)))


def _edge_body(a2_hbm, g_hbm, c_hbm, b_hbm, sl_hbm, dst_hbm, znd_hbm,
               out_hbm, acc, gsp, ctab, btab, ib, dstb, gb2, sb2, gargb,
               gs0, gs1, as0, as1, ss0, ss1):
    c = lax.axis_index("c")
    s = lax.axis_index("s")
    rbase = s * RPT
    @pl.when(s < 15)
    def _():
        pltpu.sync_copy(znd_hbm.at[pl.ds(rbase, RPT)],
                        acc.at[pl.ds(rbase, RPT)])

    @pl.when(s == 15)
    def _():
        pltpu.sync_copy(znd_hbm.at[pl.ds(15 * RPT, RPT_LAST)],
                        acc.at[pl.ds(15 * RPT, RPT_LAST)])
    pltpu.sync_copy(c_hbm.at[pl.ds(c * L, L)], ctab)
    pltpu.sync_copy(b_hbm.at[c], btab)

    @pl.when(s == 0)
    def _():
        pltpu.sync_copy(g_hbm, gsp)

    plsc.subcore_barrier()

    iota16 = lax.iota(jnp.int32, 16)
    gsems = (gs0, gs1)
    asems = (as0, as1)
    ssems = (ss0, ss1)

    def gissue(i, b):
        idx = ib.at[i, pl.ds(0, CH)]
        pltpu.async_copy(a2_hbm.at[idx], gb2.at[b], gsems[b])
        pltpu.async_copy(gsp.at[idx], gargb.at[b], asems[b])

    def gwait(b):
        idx = ib.at[0, pl.ds(0, CH)]
        pltpu.make_async_copy(a2_hbm.at[idx], gb2.at[b], gsems[b]).wait()
        pltpu.make_async_copy(gsp.at[idx], gargb.at[b], asems[b]).wait()

    def sissue(i, b):
        pltpu.async_copy(sb2.at[b], acc.at[dstb.at[i]], ssems[b], add=True)

    def swait(b):
        pltpu.make_async_copy(sb2.at[b], acc.at[dstb.at[0]], ssems[b]).wait()

    def do_par(fn, b):
        @pl.when(b == 0)
        def _():
            fn(0)

        @pl.when(b == 1)
        def _():
            fn(1)

    def super_body(u, carry):
        @pl.when(u > 0)
        def _():
            swait(0)
            swait(1)

        tidx = (c * NS + s) * NSC + u
        pltpu.sync_copy(sl_hbm.at[tidx], ib)
        pltpu.sync_copy(dst_hbm.at[tidx], dstb)
        gissue(0, 0)
        gissue(1, 1)

        def step(i, carry2):
            b = i % 2
            do_par(gwait, b)

            @pl.when(i >= 2)
            def _():
                do_par(swait, b)

            def group(g, carry3):
                l16 = ib[i, pl.ds(CH + g * 16, 16)]
                garg = gargb[b, pl.ds(g * 16, 16)]
                gate = _sigmoid(garg + plsc.load_gather(ctab, [l16]))
                for j in range(16):
                    jj = jnp.full((16,), j, jnp.int32)
                    gej = gate.at[jj].get(mode="promise_in_bounds")
                    lj = l16.at[jj].get(mode="promise_in_bounds")
                    e = g * 16 + j
                    for cb in range(8):
                        fsl = pl.ds(cb * 16, 16)
                        bia = plsc.load_gather(btab, [lj, cb * 16 + iota16])
                        sb2[b, e, fsl] = (gb2[b, e, fsl] + bia) * gej
                return carry3

            lax.fori_loop(0, CH // 16, group, 0)
            do_par(lambda bb: sissue(i, bb), b)

            @pl.when(i <= SS - 3)
            def _():
                do_par(lambda bb: gissue(i + 2, bb), b)

            return carry2

        lax.fori_loop(0, SS, step, 0)
        return carry

    lax.fori_loop(0, NSC, super_body, 0)
    swait(0)
    swait(1)
    plsc.subcore_barrier()

    @pl.when(s < 15)
    def _():
        pltpu.sync_copy(acc.at[pl.ds(rbase, RPT)],
                        out_hbm.at[c, pl.ds(rbase, RPT)])

    @pl.when(s == 15)
    def _():
        pltpu.sync_copy(acc.at[pl.ds(15 * RPT, RPT_LAST)],
                        out_hbm.at[c, pl.ds(15 * RPT, RPT_LAST)])


def _edges(a2, g_all, c_all, b_all, sl_all, dst_all, znd):
    mesh = plsc.VectorSubcoreMesh(core_axis_name="c", subcore_axis_name="s")
    f = functools.partial(
        pl.kernel,
        out_type=jax.ShapeDtypeStruct((NC, N, D), jnp.float32),
        mesh=mesh,
        scratch_types=[
            pltpu.VMEM_SHARED((N, D), jnp.float32),
            pltpu.VMEM_SHARED((2 * N,), jnp.float32),
            pltpu.VMEM((L,), jnp.float32),
            pltpu.VMEM((L, D), jnp.float32),
            pltpu.VMEM((SS, 2 * CH), jnp.int32),
            pltpu.VMEM((SS, CH), jnp.int32),
            pltpu.VMEM((2, CH, D), jnp.float32),
            pltpu.VMEM((2, CH, D), jnp.float32),
            pltpu.VMEM((2, CH), jnp.float32),
            pltpu.SemaphoreType.DMA,
            pltpu.SemaphoreType.DMA,
            pltpu.SemaphoreType.DMA,
            pltpu.SemaphoreType.DMA,
            pltpu.SemaphoreType.DMA,
            pltpu.SemaphoreType.DMA,
        ],
        compiler_params=pltpu.CompilerParams(needs_layout_passes=False),
    )(_edge_body)
    return f(a2, g_all, c_all, b_all, sl_all, dst_all, znd)


def _final_body(xl_ref, p0_ref, p1_ref, o_ref):
    o_ref[...] = jnp.maximum(xl_ref[...] + p0_ref[...] + p1_ref[...], 0.0)


def _final(x_loop, p0, p1):
    blk = pl.BlockSpec((BN, D), lambda i: (i, 0))
    return pl.pallas_call(
        _final_body,
        grid=(NB,),
        in_specs=[blk, blk, blk],
        out_specs=blk,
        out_shape=jax.ShapeDtypeStruct((N, D), jnp.float32),
    )(x_loop, p0, p1)


def kernel(x, edge_index, edge_label, W_loop, b_loop, W_loop_g, b_loop_g,
           W_dir_in, b_lab_in, W_dir_g_in, b_lab_g_in,
           W_dir_out, b_lab_out, W_dir_g_out, b_lab_g_out):
    src = edge_index[0]
    dst = edge_index[1]

    x_loop, a_in, g_in, a_out, g_out = _dense(
        x, W_loop, b_loop, W_loop_g, b_loop_g, W_dir_in, W_dir_g_in,
        W_dir_out, W_dir_g_out)

    c_in = (b_lab_in @ W_dir_g_in).reshape(L) + b_lab_g_in.reshape(L)
    c_out = (b_lab_out @ W_dir_g_out).reshape(L) + b_lab_g_out.reshape(L)

    a2 = jnp.concatenate([a_in, a_out], axis=0)
    g_all = jnp.concatenate([g_in[:, 0], g_out[:, 0]])
    c_all = jnp.concatenate([c_in, c_out])
    b_all = jnp.stack([b_lab_in, b_lab_out])
    gi = jnp.concatenate([src, dst + N])
    si = jnp.concatenate([dst, src])
    gi5 = gi.reshape(NC * NS * NSC, SS, 1, CH)
    lab5 = jnp.broadcast_to(edge_label[None], (NC, E)).reshape(
        NC * NS * NSC, SS, 1, CH)
    sl_all = jnp.concatenate([gi5, lab5], axis=2).reshape(
        NC * NS * NSC, SS, 2 * CH)
    dst_all = si.reshape(NC * NS * NSC, SS, CH)
    znd = jnp.zeros((N, D), jnp.float32)

    parts = _edges(a2, g_all, c_all, b_all, sl_all, dst_all, znd)
    return _final(x_loop, parts[0], parts[1])


# speedup vs baseline: 9.8385x; 2.5090x over previous
import functools

import jax
import jax.numpy as jnp
from jax import lax
from jax.experimental import pallas as pl
from jax.experimental.pallas import tpu as pltpu
from jax.experimental.pallas import tpu_sc as plsc

N = 10000
E = 320000
D = 128
L = 16

NC = 2
NS = 16

EPT = E // NS
CH = 80
NCH = EPT // CH
SS = 10
NSC = NCH // SS

RPT = 640
RPT_LAST = N - 15 * RPT
NB = 10
BN = N // NB


def _sigmoid(v):
    return 1.0 / (1.0 + jnp.exp(-v))


def _dense_body(x_ref, wl_ref, bl_ref, wlg_ref, blg_ref, wi_ref, wgi_ref,
                wo_ref, wgo_ref, xloop_ref, ain_ref, gin_ref, aout_ref,
                gout_ref):
    xb = x_ref[...]
    xl = jnp.dot(xb, wl_ref[...].T, preferred_element_type=jnp.float32) + bl_ref[...]
    gl = _sigmoid(jnp.dot(xl, wlg_ref[...], preferred_element_type=jnp.float32) + blg_ref[0, 0])
    xloop_ref[...] = gl * xl
    ai = jnp.dot(jnp.dot(xb, wi_ref[...], preferred_element_type=jnp.float32),
                 wi_ref[...], preferred_element_type=jnp.float32)
    ain_ref[...] = ai
    gin_ref[...] = jnp.dot(ai, wgi_ref[...], preferred_element_type=jnp.float32)
    ao = jnp.dot(jnp.dot(xb, wo_ref[...], preferred_element_type=jnp.float32),
                 wo_ref[...], preferred_element_type=jnp.float32)
    aout_ref[...] = ao
    gout_ref[...] = jnp.dot(ao, wgo_ref[...], preferred_element_type=jnp.float32)


def _dense(x, W_loop, b_loop, W_loop_g, b_loop_g, W_dir_in, W_dir_g_in,
           W_dir_out, W_dir_g_out):
    full = lambda shape: pl.BlockSpec(shape, lambda i: (0, 0))
    blk = lambda: pl.BlockSpec((BN, D), lambda i: (i, 0))
    nd = jax.ShapeDtypeStruct((N, D), jnp.float32)
    return pl.pallas_call(
        _dense_body,
        grid=(NB,),
        in_specs=[blk(), full((D, D)), full((1, D)), full((D, D)),
                  full((1, 1)), full((D, D)), full((D, D)), full((D, D)),
                  full((D, D))],
        out_specs=[blk(), blk(), blk(), blk(), blk()],
        out_shape=[nd, nd, nd, nd, nd],
    )(x, W_loop, b_loop.reshape(1, D), jnp.broadcast_to(W_loop_g.T, (D, D)),
      b_loop_g.reshape(1, 1), W_dir_in, jnp.broadcast_to(W_dir_g_in, (D, D)),
      W_dir_out, jnp.broadcast_to(W_dir_g_out, (D, D)))


def _edge_body(a2_hbm, g_hbm, c_hbm, b_hbm, sl_hbm, dst_hbm, znd_hbm,
               out_hbm, acc, gsp, ctab, btab, ib, dstb, gb2, sb2, gargb,
               gs0, gs1, as0, as1, ss0, ss1):
    c = lax.axis_index("c")
    s = lax.axis_index("s")
    rbase = s * RPT
    @pl.when(s < 15)
    def _():
        pltpu.sync_copy(znd_hbm.at[pl.ds(rbase, RPT)],
                        acc.at[pl.ds(rbase, RPT)])

    @pl.when(s == 15)
    def _():
        pltpu.sync_copy(znd_hbm.at[pl.ds(15 * RPT, RPT_LAST)],
                        acc.at[pl.ds(15 * RPT, RPT_LAST)])
    pltpu.sync_copy(c_hbm.at[pl.ds(c * L, L)], ctab)
    pltpu.sync_copy(b_hbm.at[c], btab)

    @pl.when(s == 0)
    def _():
        pltpu.sync_copy(g_hbm, gsp)

    plsc.subcore_barrier()

    iota16 = lax.iota(jnp.int32, 16)
    gsems = (gs0, gs1)
    asems = (as0, as1)
    ssems = (ss0, ss1)

    def gissue(i, b):
        idx = ib.at[i, pl.ds(0, CH)]
        pltpu.async_copy(a2_hbm.at[idx], gb2.at[b], gsems[b])
        pltpu.async_copy(gsp.at[idx], gargb.at[b], asems[b])

    def gwait(b):
        idx = ib.at[0, pl.ds(0, CH)]
        pltpu.make_async_copy(a2_hbm.at[idx], gb2.at[b], gsems[b]).wait()
        pltpu.make_async_copy(gsp.at[idx], gargb.at[b], asems[b]).wait()

    def sissue(i, b):
        pltpu.async_copy(sb2.at[b], acc.at[dstb.at[i]], ssems[b], add=True)

    def swait(b):
        pltpu.make_async_copy(sb2.at[b], acc.at[dstb.at[0]], ssems[b]).wait()

    def do_par(fn, b):
        @pl.when(b == 0)
        def _():
            fn(0)

        @pl.when(b == 1)
        def _():
            fn(1)

    def super_body(u, carry):
        @pl.when(u > 0)
        def _():
            swait(0)
            swait(1)

        tidx = (c * NS + s) * NSC + u
        pltpu.sync_copy(sl_hbm.at[tidx], ib)
        pltpu.sync_copy(dst_hbm.at[tidx], dstb)
        gissue(0, 0)
        gissue(1, 1)

        def step(i, carry2):
            b = i % 2
            do_par(gwait, b)

            @pl.when(i >= 2)
            def _():
                do_par(swait, b)

            def group(g, carry3):
                l16 = ib[i, pl.ds(CH + g * 16, 16)]
                garg = gargb[b, pl.ds(g * 16, 16)]
                gate = _sigmoid(garg + plsc.load_gather(ctab, [l16]))

                def edge(j):
                    jj = jnp.zeros((16,), jnp.int32) + j
                    gej = gate.at[jj].get(mode="promise_in_bounds")
                    lj = l16.at[jj].get(mode="promise_in_bounds")
                    e = g * 16 + j
                    for cb in range(8):
                        fsl = pl.ds(cb * 16, 16)
                        bia = plsc.load_gather(btab, [lj, cb * 16 + iota16])
                        sb2[b, e, fsl] = (gb2[b, e, fsl] + bia) * gej

                plsc.parallel_loop(0, 16, unroll=4)(edge)
                return carry3

            lax.fori_loop(0, CH // 16, group, 0)
            do_par(lambda bb: sissue(i, bb), b)

            @pl.when(i <= SS - 3)
            def _():
                do_par(lambda bb: gissue(i + 2, bb), b)

            return carry2

        lax.fori_loop(0, SS, step, 0)
        return carry

    lax.fori_loop(0, NSC, super_body, 0)
    swait(0)
    swait(1)
    plsc.subcore_barrier()

    @pl.when(s < 15)
    def _():
        pltpu.sync_copy(acc.at[pl.ds(rbase, RPT)],
                        out_hbm.at[c, pl.ds(rbase, RPT)])

    @pl.when(s == 15)
    def _():
        pltpu.sync_copy(acc.at[pl.ds(15 * RPT, RPT_LAST)],
                        out_hbm.at[c, pl.ds(15 * RPT, RPT_LAST)])


def _edges(a2, g_all, c_all, b_all, sl_all, dst_all, znd):
    mesh = plsc.VectorSubcoreMesh(core_axis_name="c", subcore_axis_name="s")
    f = functools.partial(
        pl.kernel,
        out_type=jax.ShapeDtypeStruct((NC, N, D), jnp.float32),
        mesh=mesh,
        scratch_types=[
            pltpu.VMEM_SHARED((N, D), jnp.float32),
            pltpu.VMEM_SHARED((2 * N,), jnp.float32),
            pltpu.VMEM((L,), jnp.float32),
            pltpu.VMEM((L, D), jnp.float32),
            pltpu.VMEM((SS, 2 * CH), jnp.int32),
            pltpu.VMEM((SS, CH), jnp.int32),
            pltpu.VMEM((2, CH, D), jnp.float32),
            pltpu.VMEM((2, CH, D), jnp.float32),
            pltpu.VMEM((2, CH), jnp.float32),
            pltpu.SemaphoreType.DMA,
            pltpu.SemaphoreType.DMA,
            pltpu.SemaphoreType.DMA,
            pltpu.SemaphoreType.DMA,
            pltpu.SemaphoreType.DMA,
            pltpu.SemaphoreType.DMA,
        ],
        compiler_params=pltpu.CompilerParams(needs_layout_passes=False),
    )(_edge_body)
    return f(a2, g_all, c_all, b_all, sl_all, dst_all, znd)


def _final_body(xl_ref, p0_ref, p1_ref, o_ref):
    o_ref[...] = jnp.maximum(xl_ref[...] + p0_ref[...] + p1_ref[...], 0.0)


def _final(x_loop, p0, p1):
    blk = pl.BlockSpec((BN, D), lambda i: (i, 0))
    return pl.pallas_call(
        _final_body,
        grid=(NB,),
        in_specs=[blk, blk, blk],
        out_specs=blk,
        out_shape=jax.ShapeDtypeStruct((N, D), jnp.float32),
    )(x_loop, p0, p1)


def kernel(x, edge_index, edge_label, W_loop, b_loop, W_loop_g, b_loop_g,
           W_dir_in, b_lab_in, W_dir_g_in, b_lab_g_in,
           W_dir_out, b_lab_out, W_dir_g_out, b_lab_g_out):
    src = edge_index[0]
    dst = edge_index[1]

    x_loop, a_in, g_in, a_out, g_out = _dense(
        x, W_loop, b_loop, W_loop_g, b_loop_g, W_dir_in, W_dir_g_in,
        W_dir_out, W_dir_g_out)

    c_in = (b_lab_in @ W_dir_g_in).reshape(L) + b_lab_g_in.reshape(L)
    c_out = (b_lab_out @ W_dir_g_out).reshape(L) + b_lab_g_out.reshape(L)

    a2 = jnp.concatenate([a_in, a_out], axis=0)
    g_all = jnp.concatenate([g_in[:, 0], g_out[:, 0]])
    c_all = jnp.concatenate([c_in, c_out])
    b_all = jnp.stack([b_lab_in, b_lab_out])
    gi = jnp.concatenate([src, dst + N])
    si = jnp.concatenate([dst, src])
    gi5 = gi.reshape(NC * NS * NSC, SS, 1, CH)
    lab5 = jnp.broadcast_to(edge_label[None], (NC, E)).reshape(
        NC * NS * NSC, SS, 1, CH)
    sl_all = jnp.concatenate([gi5, lab5], axis=2).reshape(
        NC * NS * NSC, SS, 2 * CH)
    dst_all = si.reshape(NC * NS * NSC, SS, CH)
    znd = jnp.zeros((N, D), jnp.float32)

    parts = _edges(a2, g_all, c_all, b_all, sl_all, dst_all, znd)
    return _final(x_loop, parts[0], parts[1])
